# baseline (speedup 1.0000x reference)
import functools

import jax
import jax.numpy as jnp
from jax import lax
from jax.experimental import pallas as pl
from jax.experimental.pallas import tpu as pltpu
from jax.experimental.pallas import tpu_sc as plsc

_D = 64
_R = 16384
_S = 50
_NC = 2
_NS = 16
_NW = _NC * _NS
_G = _R // _NW
_TB = 64
_NB = 4

_mesh = plsc.VectorSubcoreMesh(core_axis_name="c", subcore_axis_name="s")


@functools.partial(
    pl.kernel,
    mesh=_mesh,
    out_type=jax.ShapeDtypeStruct((_R, _S, _D), jnp.float32),
    compiler_params=pltpu.CompilerParams(use_tc_tiling_on_sc=False),
    scratch_types=[
        pltpu.VMEM((_TB, _S), jnp.int32),
        pltpu.VMEM((_NB, _S, _D), jnp.float32),
        pltpu.SemaphoreType.DMA((_NB,)),
        pltpu.SemaphoreType.DMA((_NB,)),
    ],
)
def _gather_all(ids_hbm, table_hbm, out_hbm, idx_v, rows_v, gsem, wsem):
    wid = lax.axis_index("s") * _NC + lax.axis_index("c")
    base = wid * _G

    def g_start(g, slot):
        pltpu.async_copy(
            table_hbm.at[idx_v.at[g % _TB]], rows_v.at[slot], gsem.at[slot])

    def g_wait(slot):
        pltpu.make_async_copy(
            table_hbm.at[idx_v.at[0]], rows_v.at[slot], gsem.at[slot]).wait()

    def w_start(g, slot):
        pltpu.async_copy(rows_v.at[slot], out_hbm.at[base + g], wsem.at[slot])

    def w_wait(slot):
        pltpu.make_async_copy(
            rows_v.at[slot], out_hbm.at[base], wsem.at[slot]).wait()

    def body(g4, carry):
        for p in range(_NB):
            g = _NB * g4 + p
            br = lax.rem(g, _TB)

            @pl.when(br == 0)
            def _():
                start = pl.multiple_of(base + g, _TB)
                pltpu.sync_copy(ids_hbm.at[pl.ds(start, _TB)], idx_v)
                for q in range(_NB - 1):
                    slot = (p + q) % _NB

                    @pl.when(g + q >= _NB)
                    def _():
                        w_wait(slot)

                    g_start(g + q, slot)

            g_wait(p)
            w_start(g, p)

            @pl.when(br < _TB - (_NB - 1))
            def _():
                nslot = (p + _NB - 1) % _NB

                @pl.when(g >= 1)
                def _():
                    w_wait(nslot)

                g_start(g + _NB - 1, nslot)
        return carry

    lax.fori_loop(0, _G // _NB, body, 0)

    for p in range(_NB):
        w_wait(p)


def kernel(token_ids, embedding):
    return _gather_all(token_ids.astype(jnp.int32), embedding)
